# Initial kernel scaffold; baseline (speedup 1.0000x reference)
import jax
import jax.numpy as jnp
from jax.experimental import pallas as pl


def kernel(heatmap, paf, candA, candB):
    raise NotImplementedError("write your pallas kernel here")


# speedup vs baseline: 7.0731x; 7.0731x over previous
import functools

import jax
import jax.numpy as jnp
import numpy as np
from jax import lax
from jax.experimental import pallas as pl
from jax.experimental.pallas import tpu as pltpu

J, P, H, W = 26, 52, 1080, 1920
K, N, MID = 26, 32, 10
THRE1, PAF_THRES = 0.1, 0.4
KSIZE, SIGMA = 11, 3.0

BAND = 216
NB = H // BAND
NPAIR = N * N
GROW = W // 128
FROWS = H * GROW


def _gauss_taps():
    x = np.arange(KSIZE, dtype=np.float32) - np.float32((KSIZE - 1) / 2.0)
    g = np.exp(-(x * x) / np.float32(2.0 * SIGMA * SIGMA)).astype(np.float32)
    g = g / g.sum()
    return [float(v) for v in g]

_TAPS = _gauss_taps()
_NEG_INF = float(np.float32(-np.inf))


def _band_mats():
    lv = np.zeros((224, 232), np.float32)
    for i in range(218):
        for dd in range(KSIZE):
            lv[i, i + dd] = np.float32(_TAPS[dd])
    r2 = np.zeros((256, 128), np.float32)
    for jp in range(128):
        for dd in range(KSIZE):
            r2[jp + dd + 3, jp] = np.float32(_TAPS[dd])
    return lv, r2

_LV, _R2 = _band_mats()


def _peaks_kernel(lv_ref, r2_ref, cur_ref, prev_ref, next_ref, out_ref):
    b = pl.program_id(1)
    cur = cur_ref[0]
    ph = prev_ref[0]
    nh = next_ref[0]

    top6_edge = jnp.concatenate(
        [cur[6 - d:7 - d] for d in range(6)], axis=0)
    top6 = jnp.where(b == 0, top6_edge, ph[2:8])
    bot6_edge = jnp.concatenate(
        [cur[214 - d:215 - d] for d in range(6)], axis=0)
    bot6 = jnp.where(b == NB - 1, bot6_edge, nh[0:6])
    ci = jnp.concatenate([top6, cur, bot6, cur[0:4]], axis=0)

    vb = jnp.dot(lv_ref[...], ci, preferred_element_type=jnp.float32)

    left8 = jnp.concatenate(
        [vb[:, 0:3]] + [vb[:, 5 - d:6 - d] for d in range(5)], axis=1)
    right = jnp.concatenate(
        [vb[:, 1918 - d:1919 - d] for d in range(5)] + [vb[:, 0:115]],
        axis=1)
    vbp = jnp.concatenate([left8, vb, right], axis=1)
    r2 = r2_ref[...]
    hb = jnp.concatenate(
        [jnp.dot(vbp[:, 128 * j:128 * j + 256], r2,
                 preferred_element_type=jnp.float32) for j in range(15)],
        axis=1)[0:218]

    ii = lax.broadcasted_iota(jnp.int32, (218, 1920), 0)
    hb = jnp.where((ii == 0) & (b == 0), _NEG_INF, hb)
    hb = jnp.where((ii == 217) & (b == NB - 1), _NEG_INF, hb)

    pv = jnp.maximum(jnp.maximum(hb[0:216], hb[1:217]), hb[2:218])
    ninf_col = jnp.full((216, 1), _NEG_INF, dtype=jnp.float32)
    pvp = jnp.concatenate([ninf_col, pv, ninf_col], axis=1)
    pooled = jnp.maximum(jnp.maximum(pvp[:, 0:1920], pvp[:, 1:1921]),
                         pvp[:, 2:1922])
    blur_c = hb[1:217]
    out_ref[0] = jnp.where(
        (blur_c == pooled) & (blur_c > jnp.float32(THRE1)), blur_c, 0.0)


def _peaks(heatmap):
    return pl.pallas_call(
        _peaks_kernel,
        grid=(J, NB),
        in_specs=[
            pl.BlockSpec((224, 232), lambda k, b: (0, 0)),
            pl.BlockSpec((256, 128), lambda k, b: (0, 0)),
            pl.BlockSpec((1, BAND, W), lambda k, b: (k, b, 0)),
            pl.BlockSpec((1, 8, W),
                         lambda k, b: (k, jnp.maximum(27 * b - 1, 0), 0)),
            pl.BlockSpec((1, 8, W),
                         lambda k, b: (k, jnp.minimum(27 * b + 27, 134), 0)),
        ],
        out_specs=pl.BlockSpec((1, BAND, W), lambda k, b: (k, b, 0)),
        out_shape=jax.ShapeDtypeStruct((J, H, W), jnp.float32),
        compiler_params=pltpu.CompilerParams(
            dimension_semantics=("parallel", "arbitrary")),
        name="peaks_blur_nms",
    )(jnp.asarray(_LV), jnp.asarray(_R2), heatmap, heatmap, heatmap)


def _paf_kernel(f0_ref, f1_ref, srow_ref, lane_ref, ux_ref, uy_ref,
                out_ref, ta, tb):
    tt = pl.program_id(1)

    def body(oo, carry):
        base = oo * 8
        rows0 = []
        rows1 = []
        for i2 in range(8):
            r = srow_ref[0, 0, base + i2]
            rows0.append(f0_ref[pl.ds(r, 1), 0, :])
            rows1.append(f1_ref[pl.ds(r, 1), 0, :])
        ta[pl.ds(pl.multiple_of(base, 8), 8), :] = jnp.concatenate(rows0, 0)
        tb[pl.ds(pl.multiple_of(base, 8), 8), :] = jnp.concatenate(rows1, 0)
        return carry

    lax.fori_loop(0, NPAIR // 8, body, 0)

    lidx = lane_ref[0]
    v0 = jnp.take_along_axis(ta[...], lidx, axis=1)
    v1 = jnp.take_along_axis(tb[...], lidx, axis=1)
    contrib = v0 * ux_ref[0] + v1 * uy_ref[0]

    @pl.when(tt == 0)
    def _():
        out_ref[0] = contrib

    @pl.when(jnp.logical_and(tt > 0, tt < MID - 1))
    def _():
        out_ref[0] = out_ref[0] + contrib

    @pl.when(tt == MID - 1)
    def _():
        m = (out_ref[0] + contrib) / jnp.float32(MID)
        m = jnp.clip(m, 0.0, 1.0)
        out_ref[0] = jnp.where(m > jnp.float32(PAF_THRES), m, 0.0)


def _paf_scores(paf, candA, candB):
    A = candA.astype(jnp.float32)
    B = candB.astype(jnp.float32)
    vec = B[:, None, :, :] - A[:, :, None, :]
    norm = jnp.sqrt(jnp.sum(vec * vec, axis=-1, keepdims=True)) + 1e-10
    u = vec / norm
    ux = u[..., 0].reshape(K, NPAIR, 1)
    uy = u[..., 1].reshape(K, NPAIR, 1)
    t = jnp.linspace(0.0, 1.0, MID)
    pts = A[:, :, None, None, :] + vec[:, :, :, None, :] * t[:, None]
    idx = pts.astype(jnp.int32)
    xi, yi = idx[..., 0], idx[..., 1]
    srow = yi * GROW + (xi // 128)
    lane = xi % 128
    srow_t = srow.transpose(0, 3, 1, 2).reshape(K * MID, 1, NPAIR)
    lane_t = lane.transpose(0, 3, 1, 2).reshape(K * MID, NPAIR, 1)
    f = paf.reshape(P * FROWS, 1, 128)

    out = pl.pallas_call(
        _paf_kernel,
        grid=(K, MID),
        in_specs=[
            pl.BlockSpec((FROWS, 1, 128), lambda k, tt: (2 * k, 0, 0)),
            pl.BlockSpec((FROWS, 1, 128), lambda k, tt: (2 * k + 1, 0, 0)),
            pl.BlockSpec((1, 1, NPAIR), lambda k, tt: (k * MID + tt, 0, 0),
                         memory_space=pltpu.SMEM),
            pl.BlockSpec((1, NPAIR, 1), lambda k, tt: (k * MID + tt, 0, 0)),
            pl.BlockSpec((1, NPAIR, 1), lambda k, tt: (k, 0, 0)),
            pl.BlockSpec((1, NPAIR, 1), lambda k, tt: (k, 0, 0)),
        ],
        out_specs=pl.BlockSpec((1, NPAIR, 1), lambda k, tt: (k, 0, 0)),
        out_shape=jax.ShapeDtypeStruct((K, NPAIR, 1), jnp.float32),
        scratch_shapes=[pltpu.VMEM((NPAIR, 128), jnp.float32),
                        pltpu.VMEM((NPAIR, 128), jnp.float32)],
        compiler_params=pltpu.CompilerParams(
            dimension_semantics=("parallel", "arbitrary")),
        name="paf_line_scores",
    )(f, f, srow_t, lane_t, ux, uy)
    return out.reshape(K, N, N)


@functools.partial(jax.jit)
def kernel(heatmap, paf, candA, candB):
    return _peaks(heatmap), _paf_scores(paf, candA, candB)


# speedup vs baseline: 7.2161x; 1.0202x over previous
import functools

import jax
import jax.numpy as jnp
import numpy as np
from jax import lax
from jax.experimental import pallas as pl
from jax.experimental.pallas import tpu as pltpu

J, P, H, W = 26, 52, 1080, 1920
K, N, MID = 26, 32, 10
THRE1, PAF_THRES = 0.1, 0.4
KSIZE, SIGMA = 11, 3.0

BAND = 216
NB = H // BAND
NPAIR = N * N
GROW = W // 128
FROWS = H * GROW


def _gauss_taps():
    x = np.arange(KSIZE, dtype=np.float32) - np.float32((KSIZE - 1) / 2.0)
    g = np.exp(-(x * x) / np.float32(2.0 * SIGMA * SIGMA)).astype(np.float32)
    g = g / g.sum()
    return [float(v) for v in g]

_TAPS = _gauss_taps()
_NEG_INF = float(np.float32(-np.inf))


def _band_mats():
    lv = np.zeros((224, 232), np.float32)
    for i in range(218):
        for dd in range(KSIZE):
            lv[i, i + dd] = np.float32(_TAPS[dd])
    r2 = np.zeros((256, 128), np.float32)
    for jp in range(128):
        for dd in range(KSIZE):
            r2[jp + dd + 3, jp] = np.float32(_TAPS[dd])
    return lv, r2

_LV, _R2 = _band_mats()


def _peaks_kernel(lv_ref, r2_ref, cur_ref, prev_ref, next_ref, out_ref):
    b = pl.program_id(1)
    cur = cur_ref[0]
    ph = prev_ref[0]
    nh = next_ref[0]

    top6_edge = jnp.concatenate(
        [cur[6 - d:7 - d] for d in range(6)], axis=0)
    top6 = jnp.where(b == 0, top6_edge, ph[2:8])
    bot6_edge = jnp.concatenate(
        [cur[214 - d:215 - d] for d in range(6)], axis=0)
    bot6 = jnp.where(b == NB - 1, bot6_edge, nh[0:6])
    ci = jnp.concatenate([top6, cur, bot6, cur[0:4]], axis=0)

    vb = jnp.dot(lv_ref[...], ci, preferred_element_type=jnp.float32)

    left8 = jnp.concatenate(
        [vb[:, 0:3]] + [vb[:, 5 - d:6 - d] for d in range(5)], axis=1)
    right = jnp.concatenate(
        [vb[:, 1918 - d:1919 - d] for d in range(5)] + [vb[:, 0:115]],
        axis=1)
    vbp = jnp.concatenate([left8, vb, right], axis=1)
    r2 = r2_ref[...]
    hb = jnp.concatenate(
        [jnp.dot(vbp[:, 128 * j:128 * j + 256], r2,
                 preferred_element_type=jnp.float32) for j in range(15)],
        axis=1)[0:218]

    ii = lax.broadcasted_iota(jnp.int32, (218, 1920), 0)
    hb = jnp.where((ii == 0) & (b == 0), _NEG_INF, hb)
    hb = jnp.where((ii == 217) & (b == NB - 1), _NEG_INF, hb)

    pv = jnp.maximum(jnp.maximum(hb[0:216], hb[1:217]), hb[2:218])
    ninf_col = jnp.full((216, 1), _NEG_INF, dtype=jnp.float32)
    pvp = jnp.concatenate([ninf_col, pv, ninf_col], axis=1)
    pooled = jnp.maximum(jnp.maximum(pvp[:, 0:1920], pvp[:, 1:1921]),
                         pvp[:, 2:1922])
    blur_c = hb[1:217]
    out_ref[0] = jnp.where(
        (blur_c == pooled) & (blur_c > jnp.float32(THRE1)), blur_c, 0.0)


def _peaks(heatmap):
    return pl.pallas_call(
        _peaks_kernel,
        grid=(J, NB),
        in_specs=[
            pl.BlockSpec((224, 232), lambda k, b: (0, 0)),
            pl.BlockSpec((256, 128), lambda k, b: (0, 0)),
            pl.BlockSpec((1, BAND, W), lambda k, b: (k, b, 0)),
            pl.BlockSpec((1, 8, W),
                         lambda k, b: (k, jnp.maximum(27 * b - 1, 0), 0)),
            pl.BlockSpec((1, 8, W),
                         lambda k, b: (k, jnp.minimum(27 * b + 27, 134), 0)),
        ],
        out_specs=pl.BlockSpec((1, BAND, W), lambda k, b: (k, b, 0)),
        out_shape=jax.ShapeDtypeStruct((J, H, W), jnp.float32),
        compiler_params=pltpu.CompilerParams(
            dimension_semantics=("parallel", "arbitrary")),
        name="peaks_blur_nms",
    )(jnp.asarray(_LV), jnp.asarray(_R2), heatmap, heatmap, heatmap)


def _paf_kernel(f0_ref, f1_ref, srow_ref, lane_ref, ux_ref, uy_ref,
                out_ref, ta, tb):
    tt = pl.program_id(1)

    def body(oo, carry):
        base = oo * 32
        rows0 = []
        rows1 = []
        for i2 in range(32):
            r = srow_ref[0, 0, base + i2]
            rows0.append(f0_ref[pl.ds(r, 1), 0, :])
            rows1.append(f1_ref[pl.ds(r, 1), 0, :])
        ta[pl.ds(pl.multiple_of(base, 8), 32), :] = jnp.concatenate(rows0, 0)
        tb[pl.ds(pl.multiple_of(base, 8), 32), :] = jnp.concatenate(rows1, 0)
        return carry

    lax.fori_loop(0, NPAIR // 32, body, 0)

    lidx = lane_ref[0]
    v0 = jnp.take_along_axis(ta[...], lidx, axis=1)
    v1 = jnp.take_along_axis(tb[...], lidx, axis=1)
    contrib = v0 * ux_ref[0] + v1 * uy_ref[0]

    @pl.when(tt == 0)
    def _():
        out_ref[0] = contrib

    @pl.when(jnp.logical_and(tt > 0, tt < MID - 1))
    def _():
        out_ref[0] = out_ref[0] + contrib

    @pl.when(tt == MID - 1)
    def _():
        m = (out_ref[0] + contrib) / jnp.float32(MID)
        m = jnp.clip(m, 0.0, 1.0)
        out_ref[0] = jnp.where(m > jnp.float32(PAF_THRES), m, 0.0)


def _paf_scores(paf, candA, candB):
    A = candA.astype(jnp.float32)
    B = candB.astype(jnp.float32)
    vec = B[:, None, :, :] - A[:, :, None, :]
    norm = jnp.sqrt(jnp.sum(vec * vec, axis=-1, keepdims=True)) + 1e-10
    u = vec / norm
    ux = u[..., 0].reshape(K, NPAIR, 1)
    uy = u[..., 1].reshape(K, NPAIR, 1)
    t = jnp.linspace(0.0, 1.0, MID)
    pts = A[:, :, None, None, :] + vec[:, :, :, None, :] * t[:, None]
    idx = pts.astype(jnp.int32)
    xi, yi = idx[..., 0], idx[..., 1]
    srow = yi * GROW + (xi // 128)
    lane = xi % 128
    srow_t = srow.transpose(0, 3, 1, 2).reshape(K * MID, 1, NPAIR)
    lane_t = lane.transpose(0, 3, 1, 2).reshape(K * MID, NPAIR, 1)
    f = paf.reshape(P * FROWS, 1, 128)

    out = pl.pallas_call(
        _paf_kernel,
        grid=(K, MID),
        in_specs=[
            pl.BlockSpec((FROWS, 1, 128), lambda k, tt: (2 * k, 0, 0)),
            pl.BlockSpec((FROWS, 1, 128), lambda k, tt: (2 * k + 1, 0, 0)),
            pl.BlockSpec((1, 1, NPAIR), lambda k, tt: (k * MID + tt, 0, 0),
                         memory_space=pltpu.SMEM),
            pl.BlockSpec((1, NPAIR, 1), lambda k, tt: (k * MID + tt, 0, 0)),
            pl.BlockSpec((1, NPAIR, 1), lambda k, tt: (k, 0, 0)),
            pl.BlockSpec((1, NPAIR, 1), lambda k, tt: (k, 0, 0)),
        ],
        out_specs=pl.BlockSpec((1, NPAIR, 1), lambda k, tt: (k, 0, 0)),
        out_shape=jax.ShapeDtypeStruct((K, NPAIR, 1), jnp.float32),
        scratch_shapes=[pltpu.VMEM((NPAIR, 128), jnp.float32),
                        pltpu.VMEM((NPAIR, 128), jnp.float32)],
        compiler_params=pltpu.CompilerParams(
            dimension_semantics=("parallel", "arbitrary")),
        name="paf_line_scores",
    )(f, f, srow_t, lane_t, ux, uy)
    return out.reshape(K, N, N)


@functools.partial(jax.jit)
def kernel(heatmap, paf, candA, candB):
    return _peaks(heatmap), _paf_scores(paf, candA, candB)
